# Initial kernel scaffold; baseline (speedup 1.0000x reference)
import jax
import jax.numpy as jnp
from jax.experimental import pallas as pl


def kernel(features, edges, edge_weight, W0, Wh, b, gamma, beta, lw):
    raise NotImplementedError("write your pallas kernel here")


# speedup vs baseline: 16.7760x; 16.7760x over previous
import functools

import numpy as np

import jax
import jax.numpy as jnp
from jax import lax
from jax.experimental import pallas as pl
from jax.experimental.pallas import tpu as pltpu
from jax.experimental.pallas import tpu_sc as plsc

N = 10000
E = 320000
IN_DIM = 128
HID = 20
L = 4
DP = 32

NC = 2
NS = 16
NW = NC * NS

NPAD = 10240
RPT = NPAD // NS

E2 = 331776
SUB = 128
CH_ROWS = 8
CHUNK = CH_ROWS * SUB
TOT_CHUNKS = E2 // CHUNK
GROUPS = CHUNK // 16

_mesh = plsc.VectorSubcoreMesh(core_axis_name="c", subcore_axis_name="s",
                               num_cores=NC, num_subcores=NS)


def _wid():
    return lax.axis_index("c") * NS + lax.axis_index("s")


@functools.partial(
    pl.kernel,
    out_type=jax.ShapeDtypeStruct((NC * NPAD,), jnp.float32),
    mesh=_mesh,
    compiler_params=pltpu.CompilerParams(needs_layout_passes=False,
                                         use_tc_tiling_on_sc=False),
    scratch_types=[
        pltpu.VMEM((CH_ROWS, SUB), jnp.int32),
        pltpu.VMEM((CH_ROWS, SUB), jnp.float32),
        pltpu.VMEM((RPT,), jnp.float32),
        pltpu.VMEM_SHARED((NPAD,), jnp.float32),
    ],
)
def _sc_degree(col_hbm, ew_hbm, deg_out, col_v, ew_v, zero_v, acc):
    c = lax.axis_index("c")
    s = lax.axis_index("s")
    wid = c * NS + s
    z16 = jnp.zeros((16,), jnp.float32)
    for i in range(RPT // 16):
        zero_v[pl.ds(i * 16, 16)] = z16
    pltpu.sync_copy(zero_v, acc.at[pl.ds(s * RPT, RPT)])
    plsc.subcore_barrier()

    def chunk_body(k, carry):
        cid = k * NW + wid
        base = cid * CH_ROWS
        pltpu.sync_copy(col_hbm.at[pl.ds(base, CH_ROWS)], col_v)
        pltpu.sync_copy(ew_hbm.at[pl.ds(base, CH_ROWS)], ew_v)
        for j in range(CH_ROWS):
            pltpu.sync_copy(ew_v.at[j], acc.at[col_v.at[j]], add=True)
        return carry

    nch = jnp.where(wid < TOT_CHUNKS % NW, TOT_CHUNKS // NW + 1,
                    TOT_CHUNKS // NW)
    lax.fori_loop(0, nch, chunk_body, 0)
    plsc.subcore_barrier()
    pltpu.sync_copy(acc.at[pl.ds(s * RPT, RPT)], zero_v)
    pltpu.sync_copy(zero_v, deg_out.at[pl.ds(c * NPAD + s * RPT, RPT)])


@functools.partial(
    pl.kernel,
    out_type=jax.ShapeDtypeStruct((E2,), jnp.float32),
    mesh=_mesh,
    compiler_params=pltpu.CompilerParams(needs_layout_passes=False,
                                         use_tc_tiling_on_sc=False),
    scratch_types=[
        pltpu.VMEM((NPAD,), jnp.float32),
        pltpu.VMEM((CH_ROWS, SUB), jnp.int32),
        pltpu.VMEM((CH_ROWS, SUB), jnp.int32),
        pltpu.VMEM((CH_ROWS, SUB), jnp.float32),
        pltpu.VMEM((CHUNK,), jnp.float32),
    ],
)
def _sc_coef(row_hbm, col_hbm, ew_hbm, dinv_hbm, c_out,
             dinv_v, row_v, col_v, ew_v, c_v):
    wid = _wid()
    pltpu.sync_copy(dinv_hbm, dinv_v)

    def chunk_body(k, carry):
        cid = k * NW + wid
        base = cid * CH_ROWS
        pltpu.sync_copy(row_hbm.at[pl.ds(base, CH_ROWS)], row_v)
        pltpu.sync_copy(col_hbm.at[pl.ds(base, CH_ROWS)], col_v)
        pltpu.sync_copy(ew_hbm.at[pl.ds(base, CH_ROWS)], ew_v)
        for j in range(CH_ROWS):
            for g in range(SUB // 16):
                r16 = row_v[j, pl.ds(g * 16, 16)]
                q16 = col_v[j, pl.ds(g * 16, 16)]
                e16 = ew_v[j, pl.ds(g * 16, 16)]
                dr = plsc.load_gather(dinv_v, [r16])
                dq = plsc.load_gather(dinv_v, [q16])
                c_v[pl.ds(j * SUB + g * 16, 16)] = dr * e16 * dq
        pltpu.sync_copy(c_v, c_out.at[pl.ds(cid * CHUNK, CHUNK)])
        return carry

    nch = jnp.where(wid < TOT_CHUNKS % NW, TOT_CHUNKS // NW + 1,
                    TOT_CHUNKS // NW)
    lax.fori_loop(0, nch, chunk_body, 0)


@functools.partial(
    pl.kernel,
    out_type=jax.ShapeDtypeStruct((NC, NPAD, DP), jnp.float32),
    mesh=_mesh,
    compiler_params=pltpu.CompilerParams(needs_layout_passes=False,
                                         use_tc_tiling_on_sc=False),
    scratch_types=[
        pltpu.VMEM((CH_ROWS, SUB), jnp.int32),
        pltpu.VMEM((CH_ROWS, SUB), jnp.int32),
        pltpu.VMEM((CHUNK,), jnp.float32),
        pltpu.VMEM((CHUNK, DP), jnp.float32),
        pltpu.VMEM((RPT, DP), jnp.float32),
        pltpu.VMEM_SHARED((NPAD, DP), jnp.float32),
        pltpu.SemaphoreType.DMA,
    ],
)
def _sc_aggregate(h_hbm, row_hbm, col_hbm, c_hbm, part_out,
                  row_v, col_v, c_v, rows_v, zero_v, acc, sem):
    c = lax.axis_index("c")
    s = lax.axis_index("s")
    wid = c * NS + s
    z16 = jnp.zeros((16,), jnp.float32)
    for i in range(RPT):
        zero_v[i, pl.ds(0, 16)] = z16
        zero_v[i, pl.ds(16, 16)] = z16
    pltpu.sync_copy(zero_v, acc.at[pl.ds(s * RPT, RPT)])
    plsc.subcore_barrier()

    def chunk_body(k, carry):
        cid = k * NW + wid
        base = cid * CH_ROWS
        pltpu.sync_copy(row_hbm.at[pl.ds(base, CH_ROWS)], row_v)
        pltpu.sync_copy(col_hbm.at[pl.ds(base, CH_ROWS)], col_v)
        pltpu.sync_copy(c_hbm.at[pl.ds(cid * CHUNK, CHUNK)], c_v)
        descs = []
        for j in range(CH_ROWS):
            descs.append(pltpu.async_copy(
                h_hbm.at[row_v.at[j]],
                rows_v.at[pl.ds(j * SUB, SUB)], sem))
        for d in descs:
            d.wait()

        def scale_body(g, carry2):
            for i in range(16):
                e = g * 16 + i
                eidx = jnp.zeros((16,), jnp.int32) + e
                ci = plsc.load_gather(c_v, [eidx])
                rows_v[e, pl.ds(0, 16)] = rows_v[e, pl.ds(0, 16)] * ci
                rows_v[e, pl.ds(16, 16)] = rows_v[e, pl.ds(16, 16)] * ci
            return carry2

        lax.fori_loop(0, GROUPS, scale_body, 0)
        for j in range(CH_ROWS):
            pltpu.sync_copy(rows_v.at[pl.ds(j * SUB, SUB)],
                            acc.at[col_v.at[j]], add=True)
        return carry

    nch = jnp.where(wid < TOT_CHUNKS % NW, TOT_CHUNKS // NW + 1,
                    TOT_CHUNKS // NW)
    lax.fori_loop(0, nch, chunk_body, 0)
    plsc.subcore_barrier()
    pltpu.sync_copy(acc.at[pl.ds(s * RPT, RPT)], zero_v)
    pltpu.sync_copy(zero_v, part_out.at[c, pl.ds(s * RPT, RPT)])


def _tc_prologue(features, w0p, degp, lwp):
    def body(f_ref, w_ref, deg_ref, lw_ref, h_ref, dinv_ref, wv_ref):
        deg = deg_ref[0] + deg_ref[1]
        dinv_ref[...] = jnp.where(deg > 0.0, lax.rsqrt(deg), 0.0)
        h_ref[...] = jnp.dot(f_ref[...], w_ref[...],
                             preferred_element_type=jnp.float32)
        lw = lw_ref[...]
        m = jnp.max(lw, axis=1, keepdims=True)
        ex = jnp.exp(lw - m)
        wv_ref[...] = ex / jnp.sum(ex, axis=1, keepdims=True)

    return pl.pallas_call(
        body,
        out_shape=[
            jax.ShapeDtypeStruct((N, DP), jnp.float32),
            jax.ShapeDtypeStruct((NPAD // 128, 128), jnp.float32),
            jax.ShapeDtypeStruct((1, 8), jnp.float32),
        ],
    )(features, w0p, degp, lwp)


def _tc_layer(li, part, bias, gam, bet, wnext, x_prev, emb_in, wv):
    has_res = li > 0
    has_next = li < L - 1

    def body(*refs):
        it = iter(refs)
        p_ref = next(it)
        b_ref = next(it)
        g_ref = next(it)
        bt_ref = next(it)
        w_ref = next(it) if has_next else None
        xp_ref = next(it) if has_res else None
        emb_ref = next(it) if has_res else None
        wv_ref = next(it)
        x_ref = next(it)
        embo_ref = next(it)
        h_ref = next(it) if has_next else None

        agg = p_ref[0, :N, :] + p_ref[1, :N, :]
        z = agg + b_ref[...]
        m = jnp.sum(z, axis=0, keepdims=True) * (1.0 / N)
        zc = z - m
        v = jnp.sum(zc * zc, axis=0, keepdims=True) * (1.0 / N)
        xn = zc * lax.rsqrt(v + 1e-5) * g_ref[...] + bt_ref[...]
        x = jnp.maximum(xn, 0.0)
        if has_res:
            x = x + 0.7 * xp_ref[...]
        x_ref[...] = x
        wl = wv_ref[0:1, li:li + 1]
        if has_res:
            embo_ref[...] = emb_ref[...] + wl * x
        else:
            embo_ref[...] = wl * x
        if has_next:
            h_ref[...] = jnp.dot(x, w_ref[...],
                                 preferred_element_type=jnp.float32)

    out_shape = [jax.ShapeDtypeStruct((N, DP), jnp.float32),
                 jax.ShapeDtypeStruct((N, DP), jnp.float32)]
    if has_next:
        out_shape.append(jax.ShapeDtypeStruct((N, DP), jnp.float32))
    args = [part, bias, gam, bet]
    if has_next:
        args.append(wnext)
    if has_res:
        args.extend([x_prev, emb_in])
    args.append(wv)
    return pl.pallas_call(body, out_shape=out_shape)(*args)


def kernel(features, edges, edge_weight, W0, Wh, b, gamma, beta, lw):
    row = edges[0]
    col = edges[1]
    pad_e = E2 - E - N
    selfn = jnp.arange(N, dtype=jnp.int32)
    row_all = jnp.concatenate(
        [row, selfn, jnp.zeros((pad_e,), jnp.int32)])
    col_all = jnp.concatenate(
        [col, selfn, N + (jnp.arange(pad_e, dtype=jnp.int32) % (NPAD - N))])
    ew_all = jnp.concatenate(
        [edge_weight, jnp.ones((N,), jnp.float32),
         jnp.zeros((pad_e,), jnp.float32)])
    row2d = row_all.reshape(E2 // SUB, SUB)
    col2d = col_all.reshape(E2 // SUB, SUB)
    ew2d = ew_all.reshape(E2 // SUB, SUB)

    w0p = jnp.pad(W0, ((0, 0), (0, DP - HID)))
    whp = jnp.pad(Wh, ((0, 0), (0, DP - HID), (0, DP - HID)))
    bp = jnp.pad(b, ((0, 0), (0, DP - HID)))
    gp = jnp.pad(gamma, ((0, 0), (0, DP - HID)))
    betap = jnp.pad(beta, ((0, 0), (0, DP - HID)))
    lwp = jnp.full((1, 8), -1e30, jnp.float32).at[0, :L].set(lw)

    degp = _sc_degree(col2d, ew2d)
    degp = degp.reshape(NC, NPAD // 128, 128)
    h0, dinv2d, wv = _tc_prologue(features, w0p, degp, lwp)
    dinv = dinv2d.reshape(NPAD)
    ce = _sc_coef(row2d, col2d, ew2d, dinv)

    x_prev = None
    emb = None
    h = h0
    for li in range(L):
        part = _sc_aggregate(h, row2d, col2d, ce)
        wnext = whp[li] if li < L - 1 else None
        outs = _tc_layer(li, part, bp[li:li + 1], gp[li:li + 1],
                         betap[li:li + 1], wnext, x_prev, emb, wv)
        if li < L - 1:
            x_prev, emb, h = outs
        else:
            x_prev, emb = outs
    return emb[:, :HID]


# speedup vs baseline: 17.8145x; 1.0619x over previous
import functools

import numpy as np

import jax
import jax.numpy as jnp
from jax import lax
from jax.experimental import pallas as pl
from jax.experimental.pallas import tpu as pltpu
from jax.experimental.pallas import tpu_sc as plsc

N = 10000
E = 320000
IN_DIM = 128
HID = 20
L = 4
DP = 32

NC = 2
NS = 16
NW = NC * NS

NPAD = 10240
RPT = NPAD // NS

E2 = 331776
SUB = 128
CH_ROWS = 8
CHUNK = CH_ROWS * SUB
TOT_CHUNKS = E2 // CHUNK
GROUPS = CHUNK // 16

_mesh = plsc.VectorSubcoreMesh(core_axis_name="c", subcore_axis_name="s",
                               num_cores=NC, num_subcores=NS)


def _wid():
    return lax.axis_index("c") * NS + lax.axis_index("s")


@functools.partial(
    pl.kernel,
    out_type=jax.ShapeDtypeStruct((NC * NPAD,), jnp.float32),
    mesh=_mesh,
    compiler_params=pltpu.CompilerParams(needs_layout_passes=False,
                                         use_tc_tiling_on_sc=False),
    scratch_types=[
        pltpu.VMEM((CH_ROWS, SUB), jnp.int32),
        pltpu.VMEM((CH_ROWS, SUB), jnp.float32),
        pltpu.VMEM((RPT,), jnp.float32),
        pltpu.VMEM_SHARED((NPAD,), jnp.float32),
    ],
)
def _sc_degree(col_hbm, ew_hbm, deg_out, col_v, ew_v, zero_v, acc):
    c = lax.axis_index("c")
    s = lax.axis_index("s")
    wid = c * NS + s
    z16 = jnp.zeros((16,), jnp.float32)
    for i in range(RPT // 16):
        zero_v[pl.ds(i * 16, 16)] = z16
    pltpu.sync_copy(zero_v, acc.at[pl.ds(s * RPT, RPT)])
    plsc.subcore_barrier()

    def chunk_body(k, carry):
        cid = k * NW + wid
        base = cid * CH_ROWS
        pltpu.sync_copy(col_hbm.at[pl.ds(base, CH_ROWS)], col_v)
        pltpu.sync_copy(ew_hbm.at[pl.ds(base, CH_ROWS)], ew_v)
        for j in range(CH_ROWS):
            pltpu.sync_copy(ew_v.at[j], acc.at[col_v.at[j]], add=True)
        return carry

    nch = jnp.where(wid < TOT_CHUNKS % NW, TOT_CHUNKS // NW + 1,
                    TOT_CHUNKS // NW)
    lax.fori_loop(0, nch, chunk_body, 0)
    plsc.subcore_barrier()
    pltpu.sync_copy(acc.at[pl.ds(s * RPT, RPT)], zero_v)
    pltpu.sync_copy(zero_v, deg_out.at[pl.ds(c * NPAD + s * RPT, RPT)])


@functools.partial(
    pl.kernel,
    out_type=jax.ShapeDtypeStruct((E2,), jnp.float32),
    mesh=_mesh,
    compiler_params=pltpu.CompilerParams(needs_layout_passes=False,
                                         use_tc_tiling_on_sc=False),
    scratch_types=[
        pltpu.VMEM((NPAD,), jnp.float32),
        pltpu.VMEM((CH_ROWS, SUB), jnp.int32),
        pltpu.VMEM((CH_ROWS, SUB), jnp.int32),
        pltpu.VMEM((CH_ROWS, SUB), jnp.float32),
        pltpu.VMEM((CHUNK,), jnp.float32),
    ],
)
def _sc_coef(row_hbm, col_hbm, ew_hbm, dinv_hbm, c_out,
             dinv_v, row_v, col_v, ew_v, c_v):
    wid = _wid()
    pltpu.sync_copy(dinv_hbm, dinv_v)

    def chunk_body(k, carry):
        cid = k * NW + wid
        base = cid * CH_ROWS
        pltpu.sync_copy(row_hbm.at[pl.ds(base, CH_ROWS)], row_v)
        pltpu.sync_copy(col_hbm.at[pl.ds(base, CH_ROWS)], col_v)
        pltpu.sync_copy(ew_hbm.at[pl.ds(base, CH_ROWS)], ew_v)
        for j in range(CH_ROWS):
            for g in range(SUB // 16):
                r16 = row_v[j, pl.ds(g * 16, 16)]
                q16 = col_v[j, pl.ds(g * 16, 16)]
                e16 = ew_v[j, pl.ds(g * 16, 16)]
                dr = plsc.load_gather(dinv_v, [r16])
                dq = plsc.load_gather(dinv_v, [q16])
                c_v[pl.ds(j * SUB + g * 16, 16)] = dr * e16 * dq
        pltpu.sync_copy(c_v, c_out.at[pl.ds(cid * CHUNK, CHUNK)])
        return carry

    nch = jnp.where(wid < TOT_CHUNKS % NW, TOT_CHUNKS // NW + 1,
                    TOT_CHUNKS // NW)
    lax.fori_loop(0, nch, chunk_body, 0)


KMAIN = TOT_CHUNKS // NW
KREM = TOT_CHUNKS % NW


@functools.partial(
    pl.kernel,
    out_type=jax.ShapeDtypeStruct((NC, NPAD, DP), jnp.float32),
    mesh=_mesh,
    compiler_params=pltpu.CompilerParams(needs_layout_passes=False,
                                         use_tc_tiling_on_sc=False),
    scratch_types=[
        pltpu.VMEM((CH_ROWS, SUB), jnp.int32),
        pltpu.VMEM((CH_ROWS, SUB), jnp.int32),
        pltpu.VMEM((CH_ROWS, SUB), jnp.int32),
        pltpu.VMEM((CH_ROWS, SUB), jnp.int32),
        pltpu.VMEM((CHUNK,), jnp.float32),
        pltpu.VMEM((CHUNK,), jnp.float32),
        pltpu.VMEM((CHUNK, DP), jnp.float32),
        pltpu.VMEM((CHUNK, DP), jnp.float32),
        pltpu.VMEM((RPT, DP), jnp.float32),
        pltpu.VMEM_SHARED((NPAD, DP), jnp.float32),
        pltpu.SemaphoreType.DMA,
    ],
)
def _sc_aggregate(h_hbm, row_hbm, col_hbm, c_hbm, part_out,
                  row_v0, row_v1, col_v0, col_v1, c_v0, c_v1,
                  rows_v0, rows_v1, zero_v, acc, sem):
    c = lax.axis_index("c")
    s = lax.axis_index("s")
    wid = c * NS + s
    row_b = (row_v0, row_v1)
    col_b = (col_v0, col_v1)
    c_b = (c_v0, c_v1)
    rows_b = (rows_v0, rows_v1)

    z16 = jnp.zeros((16,), jnp.float32)
    for i in range(RPT):
        zero_v[i, pl.ds(0, 16)] = z16
        zero_v[i, pl.ds(16, 16)] = z16
    pltpu.sync_copy(zero_v, acc.at[pl.ds(s * RPT, RPT)])
    plsc.subcore_barrier()

    def load_idx_and_gather(cid, buf):
        base = cid * CH_ROWS
        pltpu.sync_copy(row_hbm.at[pl.ds(base, CH_ROWS)], row_b[buf])
        pltpu.sync_copy(col_hbm.at[pl.ds(base, CH_ROWS)], col_b[buf])
        pltpu.sync_copy(c_hbm.at[pl.ds(cid * CHUNK, CHUNK)], c_b[buf])
        return [
            pltpu.async_copy(h_hbm.at[row_b[buf].at[j]],
                             rows_b[buf].at[pl.ds(j * SUB, SUB)], sem)
            for j in range(CH_ROWS)
        ]

    def scale(buf):
        rows_v = rows_b[buf]
        c_v = c_b[buf]

        def scale_body(g, carry2):
            for i in range(16):
                e = g * 16 + i
                eidx = jnp.zeros((16,), jnp.int32) + e
                ci = plsc.load_gather(c_v, [eidx])
                rows_v[e, pl.ds(0, 16)] = rows_v[e, pl.ds(0, 16)] * ci
                rows_v[e, pl.ds(16, 16)] = rows_v[e, pl.ds(16, 16)] * ci
            return carry2

        lax.fori_loop(0, GROUPS, scale_body, 0)

    def scatter(buf):
        for j in range(CH_ROWS):
            pltpu.sync_copy(rows_b[buf].at[pl.ds(j * SUB, SUB)],
                            acc.at[col_b[buf].at[j]], add=True)

    descs = load_idx_and_gather(wid, 0)
    for k in range(KMAIN):
        cur = k % 2
        for d in descs:
            d.wait()
        if k + 1 < KMAIN:
            descs = load_idx_and_gather((k + 1) * NW + wid, 1 - cur)
        scale(cur)
        scatter(cur)
    @pl.when(wid < KREM)
    def _():
        for d in load_idx_and_gather(KMAIN * NW + wid, 0):
            d.wait()
        scale(0)
        scatter(0)

    plsc.subcore_barrier()
    pltpu.sync_copy(acc.at[pl.ds(s * RPT, RPT)], zero_v)
    pltpu.sync_copy(zero_v, part_out.at[c, pl.ds(s * RPT, RPT)])


def _tc_prologue(features, w0p, degp, lwp):
    def body(f_ref, w_ref, deg_ref, lw_ref, h_ref, dinv_ref, wv_ref):
        deg = deg_ref[0] + deg_ref[1]
        dinv_ref[...] = jnp.where(deg > 0.0, lax.rsqrt(deg), 0.0)
        h_ref[...] = jnp.dot(f_ref[...], w_ref[...],
                             preferred_element_type=jnp.float32)
        lw = lw_ref[...]
        m = jnp.max(lw, axis=1, keepdims=True)
        ex = jnp.exp(lw - m)
        wv_ref[...] = ex / jnp.sum(ex, axis=1, keepdims=True)

    return pl.pallas_call(
        body,
        out_shape=[
            jax.ShapeDtypeStruct((N, DP), jnp.float32),
            jax.ShapeDtypeStruct((NPAD // 128, 128), jnp.float32),
            jax.ShapeDtypeStruct((1, 8), jnp.float32),
        ],
    )(features, w0p, degp, lwp)


def _tc_layer(li, part, bias, gam, bet, wnext, x_prev, emb_in, wv):
    has_res = li > 0
    has_next = li < L - 1

    def body(*refs):
        it = iter(refs)
        p_ref = next(it)
        b_ref = next(it)
        g_ref = next(it)
        bt_ref = next(it)
        w_ref = next(it) if has_next else None
        xp_ref = next(it) if has_res else None
        emb_ref = next(it) if has_res else None
        wv_ref = next(it)
        x_ref = next(it)
        embo_ref = next(it)
        h_ref = next(it) if has_next else None

        agg = p_ref[0, :N, :] + p_ref[1, :N, :]
        z = agg + b_ref[...]
        m = jnp.sum(z, axis=0, keepdims=True) * (1.0 / N)
        zc = z - m
        v = jnp.sum(zc * zc, axis=0, keepdims=True) * (1.0 / N)
        xn = zc * lax.rsqrt(v + 1e-5) * g_ref[...] + bt_ref[...]
        x = jnp.maximum(xn, 0.0)
        if has_res:
            x = x + 0.7 * xp_ref[...]
        x_ref[...] = x
        wl = wv_ref[0:1, li:li + 1]
        if has_res:
            embo_ref[...] = emb_ref[...] + wl * x
        else:
            embo_ref[...] = wl * x
        if has_next:
            h_ref[...] = jnp.dot(x, w_ref[...],
                                 preferred_element_type=jnp.float32)

    out_shape = [jax.ShapeDtypeStruct((N, DP), jnp.float32),
                 jax.ShapeDtypeStruct((N, DP), jnp.float32)]
    if has_next:
        out_shape.append(jax.ShapeDtypeStruct((N, DP), jnp.float32))
    args = [part, bias, gam, bet]
    if has_next:
        args.append(wnext)
    if has_res:
        args.extend([x_prev, emb_in])
    args.append(wv)
    return pl.pallas_call(body, out_shape=out_shape)(*args)


def kernel(features, edges, edge_weight, W0, Wh, b, gamma, beta, lw):
    row = edges[0]
    col = edges[1]
    pad_e = E2 - E - N
    selfn = jnp.arange(N, dtype=jnp.int32)
    row_all = jnp.concatenate(
        [row, selfn, jnp.zeros((pad_e,), jnp.int32)])
    col_all = jnp.concatenate(
        [col, selfn, N + (jnp.arange(pad_e, dtype=jnp.int32) % (NPAD - N))])
    ew_all = jnp.concatenate(
        [edge_weight, jnp.ones((N,), jnp.float32),
         jnp.zeros((pad_e,), jnp.float32)])
    row2d = row_all.reshape(E2 // SUB, SUB)
    col2d = col_all.reshape(E2 // SUB, SUB)
    ew2d = ew_all.reshape(E2 // SUB, SUB)

    w0p = jnp.pad(W0, ((0, 0), (0, DP - HID)))
    whp = jnp.pad(Wh, ((0, 0), (0, DP - HID), (0, DP - HID)))
    bp = jnp.pad(b, ((0, 0), (0, DP - HID)))
    gp = jnp.pad(gamma, ((0, 0), (0, DP - HID)))
    betap = jnp.pad(beta, ((0, 0), (0, DP - HID)))
    lwp = jnp.full((1, 8), -1e30, jnp.float32).at[0, :L].set(lw)

    degp = _sc_degree(col2d, ew2d)
    degp = degp.reshape(NC, NPAD // 128, 128)
    h0, dinv2d, wv = _tc_prologue(features, w0p, degp, lwp)
    dinv = dinv2d.reshape(NPAD)
    ce = _sc_coef(row2d, col2d, ew2d, dinv)

    x_prev = None
    emb = None
    h = h0
    for li in range(L):
        part = _sc_aggregate(h, row2d, col2d, ce)
        wnext = whp[li] if li < L - 1 else None
        outs = _tc_layer(li, part, bp[li:li + 1], gp[li:li + 1],
                         betap[li:li + 1], wnext, x_prev, emb, wv)
        if li < L - 1:
            x_prev, emb, h = outs
        else:
            x_prev, emb = outs
    return emb[:, :HID]


# speedup vs baseline: 19.8526x; 1.1144x over previous
import functools

import numpy as np

import jax
import jax.numpy as jnp
from jax import lax
from jax.experimental import pallas as pl
from jax.experimental.pallas import tpu as pltpu
from jax.experimental.pallas import tpu_sc as plsc

N = 10000
E = 320000
IN_DIM = 128
HID = 20
L = 4
DP = 32

NC = 2
NS = 16
NW = NC * NS

NPAD = 10240
RPT = NPAD // NS

E2 = 331776
SUB = 128
CH_ROWS = 8
CHUNK = CH_ROWS * SUB
TOT_CHUNKS = E2 // CHUNK
GROUPS = CHUNK // 16

_mesh = plsc.VectorSubcoreMesh(core_axis_name="c", subcore_axis_name="s",
                               num_cores=NC, num_subcores=NS)


def _wid():
    return lax.axis_index("s") * NC + lax.axis_index("c")


@functools.partial(
    pl.kernel,
    out_type=jax.ShapeDtypeStruct((NC * NPAD,), jnp.float32),
    mesh=_mesh,
    compiler_params=pltpu.CompilerParams(needs_layout_passes=False,
                                         use_tc_tiling_on_sc=False),
    scratch_types=[
        pltpu.VMEM((CH_ROWS, SUB), jnp.int32),
        pltpu.VMEM((CH_ROWS, SUB), jnp.float32),
        pltpu.VMEM((RPT,), jnp.float32),
        pltpu.VMEM_SHARED((NPAD,), jnp.float32),
        pltpu.SemaphoreType.DMA,
    ],
)
def _sc_degree(col_hbm, ew_hbm, deg_out, col_v, ew_v, zero_v, acc, sem):
    c = lax.axis_index("c")
    s = lax.axis_index("s")
    wid = s * NC + c
    z16 = jnp.zeros((16,), jnp.float32)
    for i in range(RPT // 16):
        zero_v[pl.ds(i * 16, 16)] = z16
    pltpu.sync_copy(zero_v, acc.at[pl.ds(s * RPT, RPT)])
    plsc.subcore_barrier()

    def chunk_body(k, carry):
        cid = k * NW + wid
        base = cid * CH_ROWS
        pltpu.sync_copy(col_hbm.at[pl.ds(base, CH_ROWS)], col_v)
        pltpu.sync_copy(ew_hbm.at[pl.ds(base, CH_ROWS)], ew_v)
        descs = [
            pltpu.async_copy(ew_v.at[j], acc.at[col_v.at[j]], sem, add=True)
            for j in range(CH_ROWS)
        ]
        for d in descs:
            d.wait()
        return carry

    nch = jnp.where(wid < TOT_CHUNKS % NW, TOT_CHUNKS // NW + 1,
                    TOT_CHUNKS // NW)
    lax.fori_loop(0, nch, chunk_body, 0)
    plsc.subcore_barrier()
    pltpu.sync_copy(acc.at[pl.ds(s * RPT, RPT)], zero_v)
    pltpu.sync_copy(zero_v, deg_out.at[pl.ds(c * NPAD + s * RPT, RPT)])


@functools.partial(
    pl.kernel,
    out_type=jax.ShapeDtypeStruct((E2,), jnp.float32),
    mesh=_mesh,
    compiler_params=pltpu.CompilerParams(needs_layout_passes=False,
                                         use_tc_tiling_on_sc=False),
    scratch_types=[
        pltpu.VMEM((NPAD,), jnp.float32),
        pltpu.VMEM((CH_ROWS, SUB), jnp.int32),
        pltpu.VMEM((CH_ROWS, SUB), jnp.int32),
        pltpu.VMEM((CH_ROWS, SUB), jnp.float32),
        pltpu.VMEM((CHUNK,), jnp.float32),
    ],
)
def _sc_coef(row_hbm, col_hbm, ew_hbm, dinv_hbm, c_out,
             dinv_v, row_v, col_v, ew_v, c_v):
    wid = _wid()
    pltpu.sync_copy(dinv_hbm, dinv_v)

    def chunk_body(k, carry):
        cid = k * NW + wid
        base = cid * CH_ROWS
        pltpu.sync_copy(row_hbm.at[pl.ds(base, CH_ROWS)], row_v)
        pltpu.sync_copy(col_hbm.at[pl.ds(base, CH_ROWS)], col_v)
        pltpu.sync_copy(ew_hbm.at[pl.ds(base, CH_ROWS)], ew_v)
        for j in range(CH_ROWS):
            for g in range(SUB // 16):
                r16 = row_v[j, pl.ds(g * 16, 16)]
                q16 = col_v[j, pl.ds(g * 16, 16)]
                e16 = ew_v[j, pl.ds(g * 16, 16)]
                dr = plsc.load_gather(dinv_v, [r16])
                dq = plsc.load_gather(dinv_v, [q16])
                c_v[pl.ds(j * SUB + g * 16, 16)] = dr * e16 * dq
        pltpu.sync_copy(c_v, c_out.at[pl.ds(cid * CHUNK, CHUNK)])
        return carry

    nch = jnp.where(wid < TOT_CHUNKS % NW, TOT_CHUNKS // NW + 1,
                    TOT_CHUNKS // NW)
    lax.fori_loop(0, nch, chunk_body, 0)


KMAIN = TOT_CHUNKS // NW
KREM = TOT_CHUNKS % NW


@functools.partial(
    pl.kernel,
    out_type=jax.ShapeDtypeStruct((NC, NPAD, DP), jnp.float32),
    mesh=_mesh,
    compiler_params=pltpu.CompilerParams(needs_layout_passes=False,
                                         use_tc_tiling_on_sc=False),
    scratch_types=[
        pltpu.VMEM((CH_ROWS, SUB), jnp.int32),
        pltpu.VMEM((CH_ROWS, SUB), jnp.int32),
        pltpu.VMEM((CH_ROWS, SUB), jnp.int32),
        pltpu.VMEM((CH_ROWS, SUB), jnp.int32),
        pltpu.VMEM((CHUNK,), jnp.float32),
        pltpu.VMEM((CHUNK,), jnp.float32),
        pltpu.VMEM((CHUNK, DP), jnp.float32),
        pltpu.VMEM((CHUNK, DP), jnp.float32),
        pltpu.VMEM((RPT, DP), jnp.float32),
        pltpu.VMEM_SHARED((NPAD, DP), jnp.float32),
        pltpu.SemaphoreType.DMA,
        pltpu.SemaphoreType.DMA,
    ],
)
def _sc_aggregate(h_hbm, row_hbm, col_hbm, c_hbm, part_out,
                  row_v0, row_v1, col_v0, col_v1, c_v0, c_v1,
                  rows_v0, rows_v1, zero_v, acc, sem, sem_s):
    c = lax.axis_index("c")
    s = lax.axis_index("s")
    wid = s * NC + c
    row_b = (row_v0, row_v1)
    col_b = (col_v0, col_v1)
    c_b = (c_v0, c_v1)
    rows_b = (rows_v0, rows_v1)

    z16 = jnp.zeros((16,), jnp.float32)
    for i in range(RPT):
        zero_v[i, pl.ds(0, 16)] = z16
        zero_v[i, pl.ds(16, 16)] = z16
    pltpu.sync_copy(zero_v, acc.at[pl.ds(s * RPT, RPT)])
    plsc.subcore_barrier()

    def load_idx_and_gather(cid, buf):
        base = cid * CH_ROWS
        pltpu.sync_copy(row_hbm.at[pl.ds(base, CH_ROWS)], row_b[buf])
        pltpu.sync_copy(col_hbm.at[pl.ds(base, CH_ROWS)], col_b[buf])
        pltpu.sync_copy(c_hbm.at[pl.ds(cid * CHUNK, CHUNK)], c_b[buf])
        return [
            pltpu.async_copy(h_hbm.at[row_b[buf].at[j]],
                             rows_b[buf].at[pl.ds(j * SUB, SUB)], sem)
            for j in range(CH_ROWS)
        ]

    def scale(buf):
        rows_v = rows_b[buf]
        c_v = c_b[buf]

        def scale_body(g, carry2):
            for i in range(16):
                e = g * 16 + i
                eidx = jnp.zeros((16,), jnp.int32) + e
                ci = plsc.load_gather(c_v, [eidx])
                rows_v[e, pl.ds(0, 16)] = rows_v[e, pl.ds(0, 16)] * ci
                rows_v[e, pl.ds(16, 16)] = rows_v[e, pl.ds(16, 16)] * ci
            return carry2

        lax.fori_loop(0, GROUPS, scale_body, 0)

    def scatter(buf):
        return [
            pltpu.async_copy(rows_b[buf].at[pl.ds(j * SUB, SUB)],
                             acc.at[col_b[buf].at[j]], sem_s, add=True)
            for j in range(CH_ROWS)
        ]

    descs = load_idx_and_gather(wid, 0)
    sdescs = []
    for k in range(KMAIN):
        cur = k % 2
        for d in descs:
            d.wait()
        for d in sdescs:
            d.wait()
        if k + 1 < KMAIN:
            descs = load_idx_and_gather((k + 1) * NW + wid, 1 - cur)
        scale(cur)
        sdescs = scatter(cur)
    for d in sdescs:
        d.wait()
    @pl.when(wid < KREM)
    def _():
        for d in load_idx_and_gather(KMAIN * NW + wid, 0):
            d.wait()
        scale(0)
        for d in scatter(0):
            d.wait()

    plsc.subcore_barrier()
    pltpu.sync_copy(acc.at[pl.ds(s * RPT, RPT)], zero_v)
    pltpu.sync_copy(zero_v, part_out.at[c, pl.ds(s * RPT, RPT)])


def _tc_prologue(features, w0p, degp, lwp):
    def body(f_ref, w_ref, deg_ref, lw_ref, h_ref, dinv_ref, wv_ref):
        deg = deg_ref[0] + deg_ref[1]
        dinv_ref[...] = jnp.where(deg > 0.0, lax.rsqrt(deg), 0.0)
        h_ref[...] = jnp.dot(f_ref[...], w_ref[...],
                             preferred_element_type=jnp.float32)
        lw = lw_ref[...]
        m = jnp.max(lw, axis=1, keepdims=True)
        ex = jnp.exp(lw - m)
        wv_ref[...] = ex / jnp.sum(ex, axis=1, keepdims=True)

    return pl.pallas_call(
        body,
        out_shape=[
            jax.ShapeDtypeStruct((N, DP), jnp.float32),
            jax.ShapeDtypeStruct((NPAD // 128, 128), jnp.float32),
            jax.ShapeDtypeStruct((1, 8), jnp.float32),
        ],
    )(features, w0p, degp, lwp)


def _tc_layer(li, part, bias, gam, bet, wnext, x_prev, emb_in, wv):
    has_res = li > 0
    has_next = li < L - 1

    def body(*refs):
        it = iter(refs)
        p_ref = next(it)
        b_ref = next(it)
        g_ref = next(it)
        bt_ref = next(it)
        w_ref = next(it) if has_next else None
        xp_ref = next(it) if has_res else None
        emb_ref = next(it) if has_res else None
        wv_ref = next(it)
        x_ref = next(it)
        embo_ref = next(it)
        h_ref = next(it) if has_next else None

        agg = p_ref[0, :N, :] + p_ref[1, :N, :]
        z = agg + b_ref[...]
        m = jnp.sum(z, axis=0, keepdims=True) * (1.0 / N)
        zc = z - m
        v = jnp.sum(zc * zc, axis=0, keepdims=True) * (1.0 / N)
        xn = zc * lax.rsqrt(v + 1e-5) * g_ref[...] + bt_ref[...]
        x = jnp.maximum(xn, 0.0)
        if has_res:
            x = x + 0.7 * xp_ref[...]
        x_ref[...] = x
        wl = wv_ref[0:1, li:li + 1]
        if has_res:
            embo_ref[...] = emb_ref[...] + wl * x
        else:
            embo_ref[...] = wl * x
        if has_next:
            h_ref[...] = jnp.dot(x, w_ref[...],
                                 preferred_element_type=jnp.float32)

    out_shape = [jax.ShapeDtypeStruct((N, DP), jnp.float32),
                 jax.ShapeDtypeStruct((N, DP), jnp.float32)]
    if has_next:
        out_shape.append(jax.ShapeDtypeStruct((N, DP), jnp.float32))
    args = [part, bias, gam, bet]
    if has_next:
        args.append(wnext)
    if has_res:
        args.extend([x_prev, emb_in])
    args.append(wv)
    return pl.pallas_call(body, out_shape=out_shape)(*args)


def kernel(features, edges, edge_weight, W0, Wh, b, gamma, beta, lw):
    row = edges[0]
    col = edges[1]
    pad_e = E2 - E - N
    selfn = jnp.arange(N, dtype=jnp.int32)
    row_all = jnp.concatenate(
        [row, selfn, jnp.zeros((pad_e,), jnp.int32)])
    col_all = jnp.concatenate(
        [col, selfn, N + (jnp.arange(pad_e, dtype=jnp.int32) % (NPAD - N))])
    ew_all = jnp.concatenate(
        [edge_weight, jnp.ones((N,), jnp.float32),
         jnp.zeros((pad_e,), jnp.float32)])
    row2d = row_all.reshape(E2 // SUB, SUB)
    col2d = col_all.reshape(E2 // SUB, SUB)
    ew2d = ew_all.reshape(E2 // SUB, SUB)

    w0p = jnp.pad(W0, ((0, 0), (0, DP - HID)))
    whp = jnp.pad(Wh, ((0, 0), (0, DP - HID), (0, DP - HID)))
    bp = jnp.pad(b, ((0, 0), (0, DP - HID)))
    gp = jnp.pad(gamma, ((0, 0), (0, DP - HID)))
    betap = jnp.pad(beta, ((0, 0), (0, DP - HID)))
    lwp = jnp.full((1, 8), -1e30, jnp.float32).at[0, :L].set(lw)

    degp = _sc_degree(col2d, ew2d)
    degp = degp.reshape(NC, NPAD // 128, 128)
    h0, dinv2d, wv = _tc_prologue(features, w0p, degp, lwp)
    dinv = dinv2d.reshape(NPAD)
    ce = _sc_coef(row2d, col2d, ew2d, dinv)

    x_prev = None
    emb = None
    h = h0
    for li in range(L):
        part = _sc_aggregate(h, row2d, col2d, ce)
        wnext = whp[li] if li < L - 1 else None
        outs = _tc_layer(li, part, bp[li:li + 1], gp[li:li + 1],
                         betap[li:li + 1], wnext, x_prev, emb, wv)
        if li < L - 1:
            x_prev, emb, h = outs
        else:
            x_prev, emb = outs
    return emb[:, :HID]
